# baseline (speedup 1.0000x reference)
import functools

import jax
import jax.numpy as jnp
from jax import lax
from jax.experimental import pallas as pl
from jax.experimental.pallas import tpu as pltpu
from jax.experimental.pallas import tpu_sc as plsc

N_FACTORS = 32
BATCH = 16384
NUM_CORES = 2
NUM_SUBCORES = 16
NUM_WORKERS = NUM_CORES * NUM_SUBCORES
LANES = 16
B_PER_W = BATCH // NUM_WORKERS
IDX_ROWS = 4
IDX_COLS = B_PER_W // IDX_ROWS
GROUPS = B_PER_W // LANES
SUB = 4

_mesh = plsc.VectorSubcoreMesh(core_axis_name="c", subcore_axis_name="s")


@functools.partial(
    pl.kernel,
    mesh=_mesh,
    out_type=jax.ShapeDtypeStruct((BATCH,), jnp.float32),
    compiler_params=pltpu.CompilerParams(needs_layout_passes=False),
    scratch_types=[
        pltpu.VMEM((IDX_ROWS, IDX_COLS), jnp.int32),
        pltpu.VMEM((IDX_ROWS, IDX_COLS), jnp.int32),
        pltpu.VMEM((2 * 2 * SUB, N_FACTORS, 128), jnp.float32),
        pltpu.VMEM((B_PER_W,), jnp.float32),
        pltpu.SemaphoreType.DMA,
        pltpu.SemaphoreType.DMA,
    ],
)
def _mf_sc(user_hbm, item_hbm, uft_hbm, vft_hbm, out_hbm,
           uidx, iidx, blk, outv, sem0, sem1):
    wid = lax.axis_index("s") * NUM_CORES + lax.axis_index("c")

    pltpu.sync_copy(user_hbm.at[pl.ds(wid * IDX_ROWS, IDX_ROWS)], uidx)
    pltpu.sync_copy(item_hbm.at[pl.ds(wid * IDX_ROWS, IDX_ROWS)], iidx)

    lanes = lax.iota(jnp.int32, LANES)
    sems = (sem0, sem1)

    def fire(uvec, vvec, j, st):
        for t in range(SUB):
            k = SUB * j + t
            ub = pl.multiple_of(jnp.bitwise_and(uvec[k], -128), 128)
            vb = pl.multiple_of(jnp.bitwise_and(vvec[k], -128), 128)
            pltpu.async_copy(uft_hbm.at[:, pl.ds(ub, 128)],
                             blk.at[st * 2 * SUB + t], sems[st])
            pltpu.async_copy(vft_hbm.at[:, pl.ds(vb, 128)],
                             blk.at[st * 2 * SUB + SUB + t], sems[st])

    def drain(st):
        for t in range(2 * SUB):
            pltpu.make_async_copy(uft_hbm.at[:, pl.ds(0, 128)],
                                  blk.at[st * 2 * SUB + t], sems[st]).wait()

    def comp(uvec, vvec, j, st, acc):
        for t in range(SUB):
            k = SUB * j + t
            ucol = jnp.broadcast_to(jnp.bitwise_and(uvec[k], 127), (LANES,))
            vcol = jnp.broadcast_to(jnp.bitwise_and(vvec[k], 127), (LANES,))
            ubr = blk.at[st * 2 * SUB + t]
            vbr = blk.at[st * 2 * SUB + SUB + t]
            u0 = plsc.load_gather(ubr, [lanes, ucol])
            u1 = plsc.load_gather(ubr, [lanes + LANES, ucol])
            v0 = plsc.load_gather(vbr, [lanes, vcol])
            v1 = plsc.load_gather(vbr, [lanes + LANES, vcol])
            p = u0 * v0 + u1 * v1
            s = jnp.sum(p)
            acc = jnp.where(lanes == k, s, acc)
        return acc

    def body(g, carry):
        r = jnp.right_shift(g, 3)
        c = pl.multiple_of(jnp.bitwise_and(g, 7) * LANES, LANES)
        uvec = uidx[r, pl.ds(c, LANES)]
        vvec = iidx[r, pl.ds(c, LANES)]

        acc = jnp.zeros((LANES,), jnp.float32)
        fire(uvec, vvec, 0, 0)
        fire(uvec, vvec, 1, 1)
        drain(0)
        acc = comp(uvec, vvec, 0, 0, acc)
        fire(uvec, vvec, 2, 0)
        drain(1)
        acc = comp(uvec, vvec, 1, 1, acc)
        fire(uvec, vvec, 3, 1)
        drain(0)
        acc = comp(uvec, vvec, 2, 0, acc)
        drain(1)
        acc = comp(uvec, vvec, 3, 1, acc)

        base = pl.multiple_of(g * LANES, LANES)
        outv[pl.ds(base, LANES)] = acc
        return carry

    lax.fori_loop(0, GROUPS, body, 0)

    pltpu.sync_copy(outv, out_hbm.at[pl.ds(wid * B_PER_W, B_PER_W)])


def kernel(user, item, user_factors, item_factors):
    u2 = user.reshape(NUM_WORKERS * IDX_ROWS, IDX_COLS)
    i2 = item.reshape(NUM_WORKERS * IDX_ROWS, IDX_COLS)
    return _mf_sc(u2, i2, user_factors.T, item_factors.T)


# speedup vs baseline: 1.0071x; 1.0071x over previous
import functools

import jax
import jax.numpy as jnp
from jax import lax
from jax.experimental import pallas as pl
from jax.experimental.pallas import tpu as pltpu
from jax.experimental.pallas import tpu_sc as plsc

N_FACTORS = 32
BATCH = 16384
NUM_CORES = 2
NUM_SUBCORES = 16
NUM_WORKERS = NUM_CORES * NUM_SUBCORES
LANES = 16
B_PER_W = BATCH // NUM_WORKERS
IDX_ROWS = 4
IDX_COLS = B_PER_W // IDX_ROWS
GROUPS = B_PER_W // LANES
SUB = 4

_mesh = plsc.VectorSubcoreMesh(core_axis_name="c", subcore_axis_name="s")


@functools.partial(
    pl.kernel,
    mesh=_mesh,
    out_type=jax.ShapeDtypeStruct((BATCH,), jnp.float32),
    compiler_params=pltpu.CompilerParams(needs_layout_passes=False),
    scratch_types=[
        pltpu.VMEM((IDX_ROWS, IDX_COLS), jnp.int32),
        pltpu.VMEM((IDX_ROWS, IDX_COLS), jnp.int32),
        pltpu.VMEM((2 * 2 * SUB, N_FACTORS, 128), jnp.float32),
        pltpu.VMEM((B_PER_W,), jnp.float32),
        pltpu.SemaphoreType.DMA,
        pltpu.SemaphoreType.DMA,
    ],
)
def _mf_sc(user_hbm, item_hbm, uft_hbm, vft_hbm, out_hbm,
           uidx, iidx, blk, outv, sem0, sem1):
    wid = lax.axis_index("s") * NUM_CORES + lax.axis_index("c")

    pltpu.sync_copy(user_hbm.at[pl.ds(wid * IDX_ROWS, IDX_ROWS)], uidx)
    pltpu.sync_copy(item_hbm.at[pl.ds(wid * IDX_ROWS, IDX_ROWS)], iidx)

    lanes = lax.iota(jnp.int32, LANES)
    sems = (sem0, sem1)

    def fire(uvec, vvec, j, st):
        for t in range(SUB):
            k = SUB * j + t
            ub = pl.multiple_of(jnp.bitwise_and(uvec[k], -128), 128)
            vb = pl.multiple_of(jnp.bitwise_and(vvec[k], -128), 128)
            for q in range(4):
                pltpu.async_copy(
                    uft_hbm.at[pl.ds(8 * q, 8), pl.ds(ub, 128)],
                    blk.at[st * 2 * SUB + t].at[pl.ds(8 * q, 8)], sems[st])
                pltpu.async_copy(
                    vft_hbm.at[pl.ds(8 * q, 8), pl.ds(vb, 128)],
                    blk.at[st * 2 * SUB + SUB + t].at[pl.ds(8 * q, 8)],
                    sems[st])

    def drain(st):
        for t in range(2 * SUB):
            pltpu.make_async_copy(uft_hbm.at[:, pl.ds(0, 128)],
                                  blk.at[st * 2 * SUB + t], sems[st]).wait()

    def comp(uvec, vvec, j, st, acc):
        for t in range(SUB):
            k = SUB * j + t
            ucol = jnp.broadcast_to(jnp.bitwise_and(uvec[k], 127), (LANES,))
            vcol = jnp.broadcast_to(jnp.bitwise_and(vvec[k], 127), (LANES,))
            ubr = blk.at[st * 2 * SUB + t]
            vbr = blk.at[st * 2 * SUB + SUB + t]
            u0 = plsc.load_gather(ubr, [lanes, ucol])
            u1 = plsc.load_gather(ubr, [lanes + LANES, ucol])
            v0 = plsc.load_gather(vbr, [lanes, vcol])
            v1 = plsc.load_gather(vbr, [lanes + LANES, vcol])
            p = u0 * v0 + u1 * v1
            s = jnp.sum(p)
            acc = jnp.where(lanes == k, s, acc)
        return acc

    def body(g, carry):
        r = jnp.right_shift(g, 3)
        c = pl.multiple_of(jnp.bitwise_and(g, 7) * LANES, LANES)
        uvec = uidx[r, pl.ds(c, LANES)]
        vvec = iidx[r, pl.ds(c, LANES)]

        acc = jnp.zeros((LANES,), jnp.float32)
        fire(uvec, vvec, 0, 0)
        fire(uvec, vvec, 1, 1)
        drain(0)
        acc = comp(uvec, vvec, 0, 0, acc)
        fire(uvec, vvec, 2, 0)
        drain(1)
        acc = comp(uvec, vvec, 1, 1, acc)
        fire(uvec, vvec, 3, 1)
        drain(0)
        acc = comp(uvec, vvec, 2, 0, acc)
        drain(1)
        acc = comp(uvec, vvec, 3, 1, acc)

        base = pl.multiple_of(g * LANES, LANES)
        outv[pl.ds(base, LANES)] = acc
        return carry

    lax.fori_loop(0, GROUPS, body, 0)

    pltpu.sync_copy(outv, out_hbm.at[pl.ds(wid * B_PER_W, B_PER_W)])


def kernel(user, item, user_factors, item_factors):
    u2 = user.reshape(NUM_WORKERS * IDX_ROWS, IDX_COLS)
    i2 = item.reshape(NUM_WORKERS * IDX_ROWS, IDX_COLS)
    return _mf_sc(u2, i2, user_factors.T, item_factors.T)


# speedup vs baseline: 1.0094x; 1.0022x over previous
import functools

import jax
import jax.numpy as jnp
from jax import lax
from jax.experimental import pallas as pl
from jax.experimental.pallas import tpu as pltpu
from jax.experimental.pallas import tpu_sc as plsc

N_FACTORS = 32
N_ROWS = 1000000
BATCH = 16384
NUM_CORES = 2
NUM_SUBCORES = 16
NUM_WORKERS = NUM_CORES * NUM_SUBCORES
LANES = 16
B_PER_W = BATCH // NUM_WORKERS
IDX_ROWS = 4
IDX_COLS = B_PER_W // IDX_ROWS
GROUPS = B_PER_W // LANES
SUB = 4

_mesh = plsc.VectorSubcoreMesh(core_axis_name="c", subcore_axis_name="s")


@functools.partial(
    pl.kernel,
    mesh=_mesh,
    out_type=jax.ShapeDtypeStruct((BATCH,), jnp.float32),
    compiler_params=pltpu.CompilerParams(needs_layout_passes=False),
    scratch_types=[
        pltpu.VMEM((IDX_ROWS, IDX_COLS), jnp.int32),
        pltpu.VMEM((IDX_ROWS, IDX_COLS), jnp.int32),
        pltpu.VMEM((2 * 2 * SUB, N_FACTORS, 128), jnp.float32),
        pltpu.VMEM((B_PER_W,), jnp.float32),
        pltpu.SemaphoreType.DMA,
        pltpu.SemaphoreType.DMA,
    ],
)
def _mf_sc(user_hbm, item_hbm, uft_hbm, vft_hbm, out_hbm,
           uidx, iidx, blk, outv, sem0, sem1):
    wid = lax.axis_index("s") * NUM_CORES + lax.axis_index("c")

    pltpu.sync_copy(user_hbm.at[pl.ds(wid * IDX_ROWS, IDX_ROWS)], uidx)
    pltpu.sync_copy(item_hbm.at[pl.ds(wid * IDX_ROWS, IDX_ROWS)], iidx)

    lanes = lax.iota(jnp.int32, LANES)
    sems = (sem0, sem1)

    def fire(uvec, vvec, j, st):
        for t in range(SUB):
            k = SUB * j + t
            ub = pl.multiple_of(jnp.bitwise_and(uvec[k], -128), 128)
            vb = pl.multiple_of(jnp.bitwise_and(vvec[k], -128), 128)
            for q in range(4):
                pltpu.async_copy(
                    uft_hbm.at[pl.ds(8 * q, 8), pl.ds(ub, 128)],
                    blk.at[st * 2 * SUB + t].at[pl.ds(8 * q, 8)], sems[st])
                pltpu.async_copy(
                    vft_hbm.at[pl.ds(8 * q, 8), pl.ds(vb, 128)],
                    blk.at[st * 2 * SUB + SUB + t].at[pl.ds(8 * q, 8)],
                    sems[st])

    def drain(st):
        for t in range(2 * SUB):
            pltpu.make_async_copy(uft_hbm.at[:, pl.ds(0, 128)],
                                  blk.at[st * 2 * SUB + t], sems[st]).wait()

    def comp(uvec, vvec, j, st, acc):
        for t in range(SUB):
            k = SUB * j + t
            ucol = jnp.broadcast_to(jnp.bitwise_and(uvec[k], 127), (LANES,))
            vcol = jnp.broadcast_to(jnp.bitwise_and(vvec[k], 127), (LANES,))
            ubr = blk.at[st * 2 * SUB + t]
            vbr = blk.at[st * 2 * SUB + SUB + t]
            u0 = plsc.load_gather(ubr, [lanes, ucol])
            u1 = plsc.load_gather(ubr, [lanes + LANES, ucol])
            v0 = plsc.load_gather(vbr, [lanes, vcol])
            v1 = plsc.load_gather(vbr, [lanes + LANES, vcol])
            p = u0 * v0 + u1 * v1
            s = jnp.sum(p)
            acc = jnp.where(lanes == k, s, acc)
        return acc

    def body(g, carry):
        r = jnp.right_shift(g, 3)
        c = pl.multiple_of(jnp.bitwise_and(g, 7) * LANES, LANES)
        uvec = uidx[r, pl.ds(c, LANES)]
        vvec = iidx[r, pl.ds(c, LANES)]

        acc = jnp.zeros((LANES,), jnp.float32)
        fire(uvec, vvec, 0, 0)
        fire(uvec, vvec, 1, 1)
        drain(0)
        acc = comp(uvec, vvec, 0, 0, acc)
        fire(uvec, vvec, 2, 0)
        drain(1)
        acc = comp(uvec, vvec, 1, 1, acc)
        fire(uvec, vvec, 3, 1)
        drain(0)
        acc = comp(uvec, vvec, 2, 0, acc)
        drain(1)
        acc = comp(uvec, vvec, 3, 1, acc)

        base = pl.multiple_of(g * LANES, LANES)
        outv[pl.ds(base, LANES)] = acc
        return carry

    lax.fori_loop(0, GROUPS, body, 0)

    pltpu.sync_copy(outv, out_hbm.at[pl.ds(wid * B_PER_W, B_PER_W)])


def kernel(user, item, user_factors, item_factors):
    u2 = user.reshape(NUM_WORKERS * IDX_ROWS, IDX_COLS)
    i2 = item.reshape(NUM_WORKERS * IDX_ROWS, IDX_COLS)
    return _mf_sc(u2, i2, user_factors.T, item_factors.T)
